# baseline (speedup 1.0000x reference)
import functools

import jax
import jax.numpy as jnp
from jax import lax
from jax.experimental import pallas as pl
from jax.experimental.pallas import tpu as pltpu
from jax.experimental.pallas import tpu_sc as plsc

N = 10000
E = 320000
D = 128
NUM_ATOM = 120
NUM_CHI = 3

NC = 2
NS = 16
NW = NC * NS
EPW = E // NW
EB = 80
C = EPW // EB
NBUF = 5
WB = 624
WBX = 8
ZB = 78

R = 1000
GRID = N // R


DW = 128


def _deg_scatter_body(dst_hbm, out_hbm, dsti_v, ones_v, acc_sh,
                      s0, s1, s2, s3, s4):
    c = lax.axis_index("c")
    s = lax.axis_index("s")
    wid = c * NS + s
    ssems = [s0, s1, s2, s3, s4]

    pltpu.async_copy(dst_hbm.at[wid], dsti_v, s0)

    @pl.loop(0, EB)
    def _zrow(i):
        for j in range(0, DW, 16):
            ones_v[i, pl.ds(j, 16)] = jnp.zeros((16,), jnp.float32)

    rbase = s * WB + jnp.minimum(s, 2) * WBX

    @pl.loop(0, WB, step=ZB)
    def _zero(r):
        pltpu.async_copy(ones_v.at[pl.ds(0, ZB)],
                         acc_sh.at[pl.ds(rbase + r, ZB)], s1)

    @pl.when(s < 2)
    def _zx():
        pltpu.sync_copy(ones_v.at[pl.ds(0, WBX)],
                        acc_sh.at[pl.ds(rbase + WB, WBX)])

    @pl.loop(0, WB, step=ZB)
    def _zdrain(r):
        pltpu.make_async_copy(ones_v.at[pl.ds(0, ZB)],
                              acc_sh.at[pl.ds(rbase + r, ZB)], s1).wait()

    @pl.loop(0, EB)
    def _frow(i):
        ones_v[i, pl.ds(0, 16)] = jnp.full((16,), 1.0, jnp.float32)

    pltpu.make_async_copy(dst_hbm.at[wid], dsti_v, s0).wait()
    plsc.subcore_barrier()

    @pl.loop(0, C, step=NBUF)
    def _edges(j):
        for b in range(NBUF):
            cur = j + b

            @pl.when(cur >= NBUF)
            def _drain():
                pltpu.make_async_copy(ones_v, acc_sh.at[dsti_v.at[cur]],
                                      ssems[b]).wait()

            pltpu.async_copy(ones_v, acc_sh.at[dsti_v.at[cur]], ssems[b],
                             add=True)

    for b in range(NBUF):
        pltpu.make_async_copy(ones_v, acc_sh.at[dsti_v.at[b]], ssems[b]).wait()

    plsc.subcore_barrier()

    pltpu.sync_copy(acc_sh.at[pl.ds(rbase, WB)],
                    out_hbm.at[c, pl.ds(rbase, WB)])

    @pl.when(s < 2)
    def _wx():
        pltpu.sync_copy(acc_sh.at[pl.ds(rbase + WB, WBX)],
                        out_hbm.at[c, pl.ds(rbase + WB, WBX)])


def _edge_scatter_body(g_hbm, src_hbm, dst_hbm, out_hbm,
                       sa0, sa1, sb0, sb1, sc0, sc1,
                       da0, da1, db0, db1, dc0, dc1,
                       r0, r1, r2, acc_sh,
                       g0, g1, g2, t0, t1, t2, i0, i1, i2):
    c = lax.axis_index("c")
    s = lax.axis_index("s")
    wid = c * NS + s
    srcs = [[sa0, sa1], [sb0, sb1], [sc0, sc1]]
    dsts = [[da0, da1], [db0, db1], [dc0, dc1]]
    rows = [r0, r1, r2]
    gsems = [g0, g1, g2]
    ssems = [t0, t1, t2]
    isems = [i0, i1, i2]
    NB = 3

    for b in range(NB):
        pltpu.async_copy(src_hbm.at[wid, pl.ds(b, 1)], srcs[b][0], isems[b])
        pltpu.async_copy(dst_hbm.at[wid, pl.ds(b, 1)], dsts[b][0], isems[b])

    @pl.loop(0, EB)
    def _zrow(i):
        for j in range(0, D, 16):
            r0[i, pl.ds(j, 16)] = jnp.zeros((16,), jnp.float32)

    rbase = s * WB + jnp.minimum(s, 2) * WBX

    @pl.loop(0, WB, step=ZB)
    def _zero(r):
        pltpu.async_copy(r0.at[pl.ds(0, ZB)], acc_sh.at[pl.ds(rbase + r, ZB)],
                         g0)

    @pl.when(s < 2)
    def _zx():
        pltpu.sync_copy(r0.at[pl.ds(0, WBX)],
                        acc_sh.at[pl.ds(rbase + WB, WBX)])

    @pl.loop(0, WB, step=ZB)
    def _zdrain(r):
        pltpu.make_async_copy(r0.at[pl.ds(0, ZB)],
                              acc_sh.at[pl.ds(rbase + r, ZB)], g0).wait()

    plsc.subcore_barrier()

    for b in range(NB):
        pltpu.make_async_copy(src_hbm.at[wid, pl.ds(b, 1)], srcs[b][0],
                              isems[b]).wait()
        pltpu.make_async_copy(dst_hbm.at[wid, pl.ds(b, 1)], dsts[b][0],
                              isems[b]).wait()
        pltpu.async_copy(g_hbm.at[srcs[b][0].at[0]], rows[b], gsems[b])

    def _maybe(cond, fn):
        if isinstance(cond, bool):
            if cond:
                fn()
        else:
            pl.when(cond)(fn)

    def _chunk(cur, b, p, guard):
        sv, dv = srcs[b][p], dsts[b][p]
        nsv, ndv = srcs[b][1 - p], dsts[b][1 - p]
        cond = (cur + NB < C) if guard else False
        nxt = jnp.int32(cur + NB) if isinstance(cur, int) else cur + NB

        def _pfi():
            pltpu.async_copy(src_hbm.at[wid, pl.ds(nxt, 1)], nsv, isems[b])
            pltpu.async_copy(dst_hbm.at[wid, pl.ds(nxt, 1)], ndv, isems[b])
        _maybe(cond, _pfi)

        pltpu.make_async_copy(g_hbm.at[sv.at[0]], rows[b], gsems[b]).wait()
        pltpu.async_copy(rows[b], acc_sh.at[dv.at[0]], ssems[b], add=True)
        pltpu.make_async_copy(rows[b], acc_sh.at[dv.at[0]], ssems[b]).wait()

        def _pfg():
            pltpu.make_async_copy(src_hbm.at[wid, pl.ds(nxt, 1)], nsv,
                                  isems[b]).wait()
            pltpu.make_async_copy(dst_hbm.at[wid, pl.ds(nxt, 1)], ndv,
                                  isems[b]).wait()
            pltpu.async_copy(g_hbm.at[nsv.at[0]], rows[b], gsems[b])
        _maybe(cond, _pfg)

    @pl.loop(0, C - 5, step=6)
    def _edges(j):
        for k in range(6):
            _chunk(j + k, k % 3, (k // 3) % 2, True)

    @pl.loop(C - 5, C - 2, step=3)
    def _edges2(j):
        for k in range(3):
            _chunk(j + k, (C - 5 + k) % 3, ((C - 5 + k) // 3) % 2, True)

    for cur in range(C - 2, C):
        _chunk(cur, cur % 3, (cur // 3) % 2, False)

    plsc.subcore_barrier()

    pltpu.sync_copy(acc_sh.at[pl.ds(rbase, WB)],
                    out_hbm.at[c, pl.ds(rbase, WB)])

    @pl.when(s < 2)
    def _wx():
        pltpu.sync_copy(acc_sh.at[pl.ds(rbase + WB, WBX)],
                        out_hbm.at[c, pl.ds(rbase + WB, WBX)])


@functools.lru_cache(maxsize=1)
def _sc_kernels():
    mesh = plsc.VectorSubcoreMesh(core_axis_name="c", subcore_axis_name="s")
    deg = pl.kernel(
        _deg_scatter_body,
        out_type=jax.ShapeDtypeStruct((NC, N, DW), jnp.float32),
        mesh=mesh,
        scratch_types=[
            pltpu.VMEM((C, EB), jnp.int32),
            pltpu.VMEM((EB, DW), jnp.float32),
            pltpu.VMEM_SHARED((N, DW), jnp.float32),
        ] + [pltpu.SemaphoreType.DMA] * NBUF,
    )
    edge = pl.kernel(
        _edge_scatter_body,
        out_type=jax.ShapeDtypeStruct((NC, N, D), jnp.float32),
        mesh=mesh,
        scratch_types=[pltpu.VMEM((1, EB), jnp.int32)] * 12
        + [pltpu.VMEM((EB, D), jnp.float32)] * 3
        + [pltpu.VMEM_SHARED((N, D), jnp.float32)]
        + [pltpu.SemaphoreType.DMA] * 9,
    )
    return deg, edge


def _dinv_block(dega_ref, degb_ref):
    deg = dega_ref[:, 0:1] + degb_ref[:, 0:1] + 1.0
    return lax.rsqrt(deg)


def _b1_body(x_ref, emb1_ref, emb2_ref, w1_ref, b1_ref, w2_ref, b2_ref,
             wc1_ref, dega_ref, degb_ref, g1_ref, xs_ref):
    xa = x_ref[:, 0:1]
    xb = x_ref[:, 1:2]
    ia = lax.broadcasted_iota(jnp.int32, (R, NUM_ATOM), 1)
    ib = lax.broadcasted_iota(jnp.int32, (R, NUM_CHI), 1)
    oa = (ia == xa).astype(jnp.float32)
    ob = (ib == xb).astype(jnp.float32)
    h0 = (jnp.dot(oa, emb1_ref[...], preferred_element_type=jnp.float32)
          + jnp.dot(ob, emb2_ref[...], preferred_element_type=jnp.float32))
    h = jnp.maximum(
        jnp.dot(h0, w1_ref[...], preferred_element_type=jnp.float32)
        + b1_ref[...], 0.0)
    g1_ref[...] = (jnp.dot(h, wc1_ref[...], preferred_element_type=jnp.float32)
                   * _dinv_block(dega_ref, degb_ref))
    xs_ref[...] = (jnp.dot(h, w2_ref[...], preferred_element_type=jnp.float32)
                   + b2_ref[...])


_b1 = pl.pallas_call(
    _b1_body,
    grid=(GRID,),
    in_specs=[
        pl.BlockSpec((R, 2), lambda i: (i, 0)),
        pl.BlockSpec((NUM_ATOM, D), lambda i: (0, 0)),
        pl.BlockSpec((NUM_CHI, D), lambda i: (0, 0)),
        pl.BlockSpec((D, D), lambda i: (0, 0)),
        pl.BlockSpec((1, D), lambda i: (0, 0)),
        pl.BlockSpec((D, D), lambda i: (0, 0)),
        pl.BlockSpec((1, D), lambda i: (0, 0)),
        pl.BlockSpec((D, D), lambda i: (0, 0)),
        pl.BlockSpec((R, DW), lambda i: (i, 0)),
        pl.BlockSpec((R, DW), lambda i: (i, 0)),
    ],
    out_specs=[
        pl.BlockSpec((R, D), lambda i: (i, 0)),
        pl.BlockSpec((R, D), lambda i: (i, 0)),
    ],
    out_shape=[
        jax.ShapeDtypeStruct((N, D), jnp.float32),
        jax.ShapeDtypeStruct((N, D), jnp.float32),
    ],
)


def _mid_body(s1a_ref, s1b_ref, g1_ref, dega_ref, degb_ref, xs_ref,
              bc1_ref, wc2_ref, g2_ref):
    dinv = _dinv_block(dega_ref, degb_ref)
    conv = dinv * (s1a_ref[...] + s1b_ref[...] + g1_ref[...]) + bc1_ref[...]
    h1 = jnp.maximum(conv, 0.0) + xs_ref[...]
    g2_ref[...] = jnp.dot(h1, wc2_ref[...],
                          preferred_element_type=jnp.float32) * dinv


_mid = pl.pallas_call(
    _mid_body,
    grid=(GRID,),
    in_specs=[
        pl.BlockSpec((R, D), lambda i: (i, 0)),
        pl.BlockSpec((R, D), lambda i: (i, 0)),
        pl.BlockSpec((R, D), lambda i: (i, 0)),
        pl.BlockSpec((R, DW), lambda i: (i, 0)),
        pl.BlockSpec((R, DW), lambda i: (i, 0)),
        pl.BlockSpec((R, D), lambda i: (i, 0)),
        pl.BlockSpec((1, D), lambda i: (0, 0)),
        pl.BlockSpec((D, D), lambda i: (0, 0)),
    ],
    out_specs=pl.BlockSpec((R, D), lambda i: (i, 0)),
    out_shape=jax.ShapeDtypeStruct((N, D), jnp.float32),
)


def _fin_body(s2a_ref, s2b_ref, g2_ref, dega_ref, degb_ref, xs_ref,
              bc2_ref, out_ref):
    dinv = _dinv_block(dega_ref, degb_ref)
    conv = dinv * (s2a_ref[...] + s2b_ref[...] + g2_ref[...]) + bc2_ref[...]
    h2 = jnp.maximum(conv, 0.0) + xs_ref[...]
    part = jnp.sum(h2, axis=0, keepdims=True) * (1.0 / N)

    @pl.when(pl.program_id(0) == 0)
    def _init():
        out_ref[...] = part

    @pl.when(pl.program_id(0) != 0)
    def _acc():
        out_ref[...] = out_ref[...] + part


_fin = pl.pallas_call(
    _fin_body,
    grid=(GRID,),
    in_specs=[
        pl.BlockSpec((R, D), lambda i: (i, 0)),
        pl.BlockSpec((R, D), lambda i: (i, 0)),
        pl.BlockSpec((R, D), lambda i: (i, 0)),
        pl.BlockSpec((R, DW), lambda i: (i, 0)),
        pl.BlockSpec((R, DW), lambda i: (i, 0)),
        pl.BlockSpec((R, D), lambda i: (i, 0)),
        pl.BlockSpec((1, D), lambda i: (0, 0)),
    ],
    out_specs=pl.BlockSpec((1, D), lambda i: (0, 0)),
    out_shape=jax.ShapeDtypeStruct((1, D), jnp.float32),
)


def kernel(x, edge_index, emb1, emb2, W1, b1, W2, b2, Wc1, bc1, Wc2, bc2):
    src = edge_index[0]
    dst = edge_index[1]
    b1r = b1.reshape(1, D)
    b2r = b2.reshape(1, D)
    bc1r = bc1.reshape(1, D)
    bc2r = bc2.reshape(1, D)

    _deg_scatter, _edge_scatter = _sc_kernels()
    src3 = src.reshape(NW, C, EB)
    dst3 = dst.reshape(NW, C, EB)
    degp = _deg_scatter(dst3)
    dega, degb = degp[0], degp[1]

    g1, xs = _b1(x, emb1, emb2, W1, b1r, W2, b2r, Wc1, dega, degb)

    s1 = _edge_scatter(g1, src3, dst3)
    g2 = _mid(s1[0], s1[1], g1, dega, degb, xs, bc1r, Wc2)

    s2 = _edge_scatter(g2, src3, dst3)
    out = _fin(s2[0], s2[1], g2, dega, degb, xs, bc2r)
    return out.reshape(D)


# speedup vs baseline: 1.0152x; 1.0152x over previous
import functools

import jax
import jax.numpy as jnp
from jax import lax
from jax.experimental import pallas as pl
from jax.experimental.pallas import tpu as pltpu
from jax.experimental.pallas import tpu_sc as plsc

N = 10000
E = 320000
D = 128
NUM_ATOM = 120
NUM_CHI = 3

NC = 2
NS = 16
NW = NC * NS
EPW = E // NW
EB = 80
C = EPW // EB
NBUF = 5
WB = 624
WBX = 8
ZB = 78

R = 1000
GRID = N // R


DW = 128


def _deg_scatter_body(dst_hbm, out_hbm, dsti_v, ones_v, acc_sh,
                      s0, s1, s2, s3, s4):
    c = lax.axis_index("c")
    s = lax.axis_index("s")
    wid = c * NS + s
    ssems = [s0, s1, s2, s3, s4]

    pltpu.async_copy(dst_hbm.at[wid], dsti_v, s0)

    @pl.loop(0, EB)
    def _zrow(i):
        for j in range(0, DW, 16):
            ones_v[i, pl.ds(j, 16)] = jnp.zeros((16,), jnp.float32)

    rbase = s * WB + jnp.minimum(s, 2) * WBX

    @pl.loop(0, WB, step=ZB)
    def _zero(r):
        pltpu.async_copy(ones_v.at[pl.ds(0, ZB)],
                         acc_sh.at[pl.ds(rbase + r, ZB)], s1)

    @pl.when(s < 2)
    def _zx():
        pltpu.sync_copy(ones_v.at[pl.ds(0, WBX)],
                        acc_sh.at[pl.ds(rbase + WB, WBX)])

    @pl.loop(0, WB, step=ZB)
    def _zdrain(r):
        pltpu.make_async_copy(ones_v.at[pl.ds(0, ZB)],
                              acc_sh.at[pl.ds(rbase + r, ZB)], s1).wait()

    @pl.loop(0, EB)
    def _frow(i):
        ones_v[i, pl.ds(0, 16)] = jnp.full((16,), 1.0, jnp.float32)

    pltpu.make_async_copy(dst_hbm.at[wid], dsti_v, s0).wait()
    plsc.subcore_barrier()

    @pl.loop(0, C, step=NBUF)
    def _edges(j):
        for b in range(NBUF):
            cur = j + b

            @pl.when(cur >= NBUF)
            def _drain():
                pltpu.make_async_copy(ones_v, acc_sh.at[dsti_v.at[cur]],
                                      ssems[b]).wait()

            pltpu.async_copy(ones_v, acc_sh.at[dsti_v.at[cur]], ssems[b],
                             add=True)

    for b in range(NBUF):
        pltpu.make_async_copy(ones_v, acc_sh.at[dsti_v.at[b]], ssems[b]).wait()

    plsc.subcore_barrier()

    pltpu.sync_copy(acc_sh.at[pl.ds(rbase, WB)],
                    out_hbm.at[c, pl.ds(rbase, WB)])

    @pl.when(s < 2)
    def _wx():
        pltpu.sync_copy(acc_sh.at[pl.ds(rbase + WB, WBX)],
                        out_hbm.at[c, pl.ds(rbase + WB, WBX)])


def _edge_scatter_body(g_hbm, src_hbm, dst_hbm, out_hbm,
                       sa0, sa1, sb0, sb1, sc0, sc1,
                       da0, da1, db0, db1, dc0, dc1,
                       r0, r1, r2, acc_sh,
                       g0, g1, g2, t0, t1, t2, i0, i1, i2):
    c = lax.axis_index("c")
    s = lax.axis_index("s")
    wid = c * NS + s
    srcs = [[sa0, sa1], [sb0, sb1], [sc0, sc1]]
    dsts = [[da0, da1], [db0, db1], [dc0, dc1]]
    rows = [r0, r1, r2]
    gsems = [g0, g1, g2]
    ssems = [t0, t1, t2]
    isems = [i0, i1, i2]
    NB = 3

    for b in range(NB):
        pltpu.async_copy(src_hbm.at[wid, pl.ds(b, 1)], srcs[b][0], isems[b])
        pltpu.async_copy(dst_hbm.at[wid, pl.ds(b, 1)], dsts[b][0], isems[b])

    @pl.loop(0, EB)
    def _zrow(i):
        for j in range(0, D, 16):
            r0[i, pl.ds(j, 16)] = jnp.zeros((16,), jnp.float32)

    rbase = s * WB + jnp.minimum(s, 2) * WBX

    @pl.loop(0, WB, step=ZB)
    def _zero(r):
        pltpu.async_copy(r0.at[pl.ds(0, ZB)], acc_sh.at[pl.ds(rbase + r, ZB)],
                         g0)

    @pl.when(s < 2)
    def _zx():
        pltpu.sync_copy(r0.at[pl.ds(0, WBX)],
                        acc_sh.at[pl.ds(rbase + WB, WBX)])

    @pl.loop(0, WB, step=ZB)
    def _zdrain(r):
        pltpu.make_async_copy(r0.at[pl.ds(0, ZB)],
                              acc_sh.at[pl.ds(rbase + r, ZB)], g0).wait()

    plsc.subcore_barrier()

    for b in range(NB):
        pltpu.make_async_copy(src_hbm.at[wid, pl.ds(b, 1)], srcs[b][0],
                              isems[b]).wait()
        pltpu.make_async_copy(dst_hbm.at[wid, pl.ds(b, 1)], dsts[b][0],
                              isems[b]).wait()
        pltpu.async_copy(g_hbm.at[srcs[b][0].at[0]], rows[b], gsems[b])

    def _maybe(cond, fn):
        if isinstance(cond, bool):
            if cond:
                fn()
        else:
            pl.when(cond)(fn)

    def _chunk(cur, b, p, guard):
        sv, dv = srcs[b][p], dsts[b][p]
        nsv, ndv = srcs[b][1 - p], dsts[b][1 - p]
        cond = (cur + NB < C) if guard else False
        nxt = jnp.int32(cur + NB) if isinstance(cur, int) else cur + NB

        def _pfi():
            pltpu.async_copy(src_hbm.at[wid, pl.ds(nxt, 1)], nsv, isems[b])
            pltpu.async_copy(dst_hbm.at[wid, pl.ds(nxt, 1)], ndv, isems[b])
        _maybe(cond, _pfi)

        pltpu.make_async_copy(g_hbm.at[sv.at[0]], rows[b], gsems[b]).wait()
        pltpu.async_copy(rows[b], acc_sh.at[dv.at[0]], ssems[b], add=True)
        pltpu.make_async_copy(rows[b], acc_sh.at[dv.at[0]], ssems[b]).wait()

        def _pfg():
            pltpu.make_async_copy(src_hbm.at[wid, pl.ds(nxt, 1)], nsv,
                                  isems[b]).wait()
            pltpu.make_async_copy(dst_hbm.at[wid, pl.ds(nxt, 1)], ndv,
                                  isems[b]).wait()
            pltpu.async_copy(g_hbm.at[nsv.at[0]], rows[b], gsems[b])
        _maybe(cond, _pfg)

    @pl.loop(0, C - 5, step=6)
    def _edges(j):
        for k in range(6):
            _chunk(j + k, k % 3, (k // 3) % 2, True)

    @pl.loop(C - 5, C - 2, step=3)
    def _edges2(j):
        for k in range(3):
            _chunk(j + k, (C - 5 + k) % 3, ((C - 5 + k) // 3) % 2, True)

    for cur in range(C - 2, C):
        _chunk(cur, cur % 3, (cur // 3) % 2, False)

    plsc.subcore_barrier()

    pltpu.sync_copy(acc_sh.at[pl.ds(rbase, WB)],
                    out_hbm.at[c, pl.ds(rbase, WB)])

    @pl.when(s < 2)
    def _wx():
        pltpu.sync_copy(acc_sh.at[pl.ds(rbase + WB, WBX)],
                        out_hbm.at[c, pl.ds(rbase + WB, WBX)])


@functools.lru_cache(maxsize=1)
def _sc_kernels():
    mesh = plsc.VectorSubcoreMesh(core_axis_name="c", subcore_axis_name="s")
    deg = pl.kernel(
        _deg_scatter_body,
        out_type=jax.ShapeDtypeStruct((NC, N, DW), jnp.float32),
        mesh=mesh,
        scratch_types=[
            pltpu.VMEM((C, EB), jnp.int32),
            pltpu.VMEM((EB, DW), jnp.float32),
            pltpu.VMEM_SHARED((N, DW), jnp.float32),
        ] + [pltpu.SemaphoreType.DMA] * NBUF,
    )
    edge = pl.kernel(
        _edge_scatter_body,
        out_type=jax.ShapeDtypeStruct((NC, N, D), jnp.float32),
        mesh=mesh,
        scratch_types=[pltpu.VMEM((1, EB), jnp.int32)] * 12
        + [pltpu.VMEM((EB, D), jnp.float32)] * 3
        + [pltpu.VMEM_SHARED((N, D), jnp.float32)]
        + [pltpu.SemaphoreType.DMA] * 9,
    )
    return deg, edge


def _dinv_block(dega_ref, degb_ref):
    deg = dega_ref[:, 0:1] + degb_ref[:, 0:1] + 1.0
    return lax.rsqrt(deg)


def _b1_body(x_ref, emb1_ref, emb2_ref, w1_ref, b1_ref, w2_ref, b2_ref,
             wc1_ref, hw1_ref, xs_ref):
    xa = x_ref[:, 0:1]
    xb = x_ref[:, 1:2]
    ia = lax.broadcasted_iota(jnp.int32, (R, NUM_ATOM), 1)
    ib = lax.broadcasted_iota(jnp.int32, (R, NUM_CHI), 1)
    oa = (ia == xa).astype(jnp.float32)
    ob = (ib == xb).astype(jnp.float32)
    h0 = (jnp.dot(oa, emb1_ref[...], preferred_element_type=jnp.float32)
          + jnp.dot(ob, emb2_ref[...], preferred_element_type=jnp.float32))
    h = jnp.maximum(
        jnp.dot(h0, w1_ref[...], preferred_element_type=jnp.float32)
        + b1_ref[...], 0.0)
    hw1_ref[...] = jnp.dot(h, wc1_ref[...], preferred_element_type=jnp.float32)
    xs_ref[...] = (jnp.dot(h, w2_ref[...], preferred_element_type=jnp.float32)
                   + b2_ref[...])


_b1 = pl.pallas_call(
    _b1_body,
    grid=(GRID,),
    in_specs=[
        pl.BlockSpec((R, 2), lambda i: (i, 0)),
        pl.BlockSpec((NUM_ATOM, D), lambda i: (0, 0)),
        pl.BlockSpec((NUM_CHI, D), lambda i: (0, 0)),
        pl.BlockSpec((D, D), lambda i: (0, 0)),
        pl.BlockSpec((1, D), lambda i: (0, 0)),
        pl.BlockSpec((D, D), lambda i: (0, 0)),
        pl.BlockSpec((1, D), lambda i: (0, 0)),
        pl.BlockSpec((D, D), lambda i: (0, 0)),
    ],
    out_specs=[
        pl.BlockSpec((R, D), lambda i: (i, 0)),
        pl.BlockSpec((R, D), lambda i: (i, 0)),
    ],
    out_shape=[
        jax.ShapeDtypeStruct((N, D), jnp.float32),
        jax.ShapeDtypeStruct((N, D), jnp.float32),
    ],
)


def _b2_body(dega_ref, degb_ref, hw1_ref, g1_ref):
    g1_ref[...] = hw1_ref[...] * _dinv_block(dega_ref, degb_ref)


_b2 = pl.pallas_call(
    _b2_body,
    grid=(GRID,),
    in_specs=[
        pl.BlockSpec((R, DW), lambda i: (i, 0)),
        pl.BlockSpec((R, DW), lambda i: (i, 0)),
        pl.BlockSpec((R, D), lambda i: (i, 0)),
    ],
    out_specs=pl.BlockSpec((R, D), lambda i: (i, 0)),
    out_shape=jax.ShapeDtypeStruct((N, D), jnp.float32),
)


def _mid_body(s1a_ref, s1b_ref, g1_ref, dega_ref, degb_ref, xs_ref,
              bc1_ref, wc2_ref, g2_ref):
    dinv = _dinv_block(dega_ref, degb_ref)
    conv = dinv * (s1a_ref[...] + s1b_ref[...] + g1_ref[...]) + bc1_ref[...]
    h1 = jnp.maximum(conv, 0.0) + xs_ref[...]
    g2_ref[...] = jnp.dot(h1, wc2_ref[...],
                          preferred_element_type=jnp.float32) * dinv


_mid = pl.pallas_call(
    _mid_body,
    grid=(GRID,),
    in_specs=[
        pl.BlockSpec((R, D), lambda i: (i, 0)),
        pl.BlockSpec((R, D), lambda i: (i, 0)),
        pl.BlockSpec((R, D), lambda i: (i, 0)),
        pl.BlockSpec((R, DW), lambda i: (i, 0)),
        pl.BlockSpec((R, DW), lambda i: (i, 0)),
        pl.BlockSpec((R, D), lambda i: (i, 0)),
        pl.BlockSpec((1, D), lambda i: (0, 0)),
        pl.BlockSpec((D, D), lambda i: (0, 0)),
    ],
    out_specs=pl.BlockSpec((R, D), lambda i: (i, 0)),
    out_shape=jax.ShapeDtypeStruct((N, D), jnp.float32),
)


def _fin_body(s2a_ref, s2b_ref, g2_ref, dega_ref, degb_ref, xs_ref,
              bc2_ref, out_ref):
    dinv = _dinv_block(dega_ref, degb_ref)
    conv = dinv * (s2a_ref[...] + s2b_ref[...] + g2_ref[...]) + bc2_ref[...]
    h2 = jnp.maximum(conv, 0.0) + xs_ref[...]
    part = jnp.sum(h2, axis=0, keepdims=True) * (1.0 / N)

    @pl.when(pl.program_id(0) == 0)
    def _init():
        out_ref[...] = part

    @pl.when(pl.program_id(0) != 0)
    def _acc():
        out_ref[...] = out_ref[...] + part


_fin = pl.pallas_call(
    _fin_body,
    grid=(GRID,),
    in_specs=[
        pl.BlockSpec((R, D), lambda i: (i, 0)),
        pl.BlockSpec((R, D), lambda i: (i, 0)),
        pl.BlockSpec((R, D), lambda i: (i, 0)),
        pl.BlockSpec((R, DW), lambda i: (i, 0)),
        pl.BlockSpec((R, DW), lambda i: (i, 0)),
        pl.BlockSpec((R, D), lambda i: (i, 0)),
        pl.BlockSpec((1, D), lambda i: (0, 0)),
    ],
    out_specs=pl.BlockSpec((1, D), lambda i: (0, 0)),
    out_shape=jax.ShapeDtypeStruct((1, D), jnp.float32),
)


def kernel(x, edge_index, emb1, emb2, W1, b1, W2, b2, Wc1, bc1, Wc2, bc2):
    src = edge_index[0]
    dst = edge_index[1]
    b1r = b1.reshape(1, D)
    b2r = b2.reshape(1, D)
    bc1r = bc1.reshape(1, D)
    bc2r = bc2.reshape(1, D)

    _deg_scatter, _edge_scatter = _sc_kernels()
    src3 = src.reshape(NW, C, EB)
    dst3 = dst.reshape(NW, C, EB)
    degp = _deg_scatter(dst3)
    dega, degb = degp[0], degp[1]

    hw1, xs = _b1(x, emb1, emb2, W1, b1r, W2, b2r, Wc1)
    g1 = _b2(dega, degb, hw1)

    s1 = _edge_scatter(g1, src3, dst3)
    g2 = _mid(s1[0], s1[1], g1, dega, degb, xs, bc1r, Wc2)

    s2 = _edge_scatter(g2, src3, dst3)
    out = _fin(s2[0], s2[1], g2, dega, degb, xs, bc2r)
    return out.reshape(D)
